# baseline (device time: 22275 ns/iter reference)
import jax
import jax.numpy as jnp
from jax import lax
from jax.experimental import pallas as pl
from jax.experimental.pallas import tpu as pltpu

C = 16
T = 3


def kernel(x):
    m, n = x.shape
    half = m // 2
    chunk = half // C

    def body(x_ref, out_ref, xs, xr, ys, yr, own_sem):
        my_x = lax.axis_index("x")
        my_y = lax.axis_index("y")
        other_x = 1 - my_x
        other_y = 1 - my_y

        barrier = pltpu.get_barrier_semaphore()
        for nbr in ((other_x, my_y), (my_x, other_y)):
            pl.semaphore_signal(
                barrier, inc=1, device_id=nbr,
                device_id_type=pl.DeviceIdType.MESH,
            )
        pl.semaphore_wait(barrier, 2)

        x_rdmas = []
        for c in range(C):
            r = pltpu.make_async_remote_copy(
                src_ref=x_ref.at[pl.ds(my_y * half + c * chunk, chunk), :],
                dst_ref=out_ref.at[pl.ds(my_x * m + my_y * half + c * chunk, chunk), :],
                send_sem=xs.at[c],
                recv_sem=xr.at[c],
                device_id=(other_x, my_y),
                device_id_type=pl.DeviceIdType.MESH,
            )
            r.start()
            x_rdmas.append(r)
        for t in range(T):
            c = C - T + t
            r = pltpu.make_async_remote_copy(
                src_ref=x_ref.at[pl.ds(other_y * half + c * chunk, chunk), :],
                dst_ref=out_ref.at[pl.ds(my_x * m + other_y * half + c * chunk, chunk), :],
                send_sem=xs.at[C + t],
                recv_sem=xr.at[C + t],
                device_id=(other_x, my_y),
                device_id_type=pl.DeviceIdType.MESH,
            )
            r.start()
            x_rdmas.append(r)

        y_rdmas = []
        for c in range(C - T):
            x_rdmas[c].wait_recv()
            off = other_x * m + my_y * half + c * chunk
            r = pltpu.make_async_remote_copy(
                src_ref=out_ref.at[pl.ds(off, chunk), :],
                dst_ref=out_ref.at[pl.ds(off, chunk), :],
                send_sem=ys.at[c],
                recv_sem=yr.at[c],
                device_id=(my_x, other_y),
                device_id_type=pl.DeviceIdType.MESH,
            )
            r.start()
            y_rdmas.append(r)

        own = pltpu.make_async_copy(
            x_ref, out_ref.at[pl.ds(my_x * m, m), :], own_sem
        )
        own.start()

        for c in range(C - T, C + T):
            x_rdmas[c].wait_recv()
        for c in range(C - T):
            y_rdmas[c].wait_recv()
        own.wait()
        for r in x_rdmas:
            r.wait_send()
        for r in y_rdmas:
            r.wait_send()

    return pl.pallas_call(
        body,
        out_shape=jax.ShapeDtypeStruct((2 * m, n), x.dtype),
        in_specs=[pl.BlockSpec(memory_space=pltpu.HBM)],
        out_specs=pl.BlockSpec(memory_space=pltpu.VMEM)    ,
        scratch_shapes=[
            pltpu.SemaphoreType.DMA((C + T,)),
            pltpu.SemaphoreType.DMA((C + T,)),
            pltpu.SemaphoreType.DMA((C - T,)),
            pltpu.SemaphoreType.DMA((C - T,)),
            pltpu.SemaphoreType.DMA,
        ],
        compiler_params=pltpu.CompilerParams(collective_id=0),
    )(x)


# device time: 22098 ns/iter; 1.0080x vs baseline; 1.0080x over previous
import jax
import jax.numpy as jnp
from jax import lax
from jax.experimental import pallas as pl
from jax.experimental.pallas import tpu as pltpu

C = 16
T = 0


def kernel(x):
    m, n = x.shape
    half = m // 2
    chunk = half // C

    def body(x_ref, out_ref, xs, xr, ys, yr, own_sem):
        my_x = lax.axis_index("x")
        my_y = lax.axis_index("y")
        other_x = 1 - my_x
        other_y = 1 - my_y

        barrier = pltpu.get_barrier_semaphore()
        for nbr in ((other_x, my_y), (my_x, other_y)):
            pl.semaphore_signal(
                barrier, inc=1, device_id=nbr,
                device_id_type=pl.DeviceIdType.MESH,
            )
        pl.semaphore_wait(barrier, 2)

        x_rdmas = []
        for c in range(C):
            r = pltpu.make_async_remote_copy(
                src_ref=x_ref.at[pl.ds(my_y * half + c * chunk, chunk), :],
                dst_ref=out_ref.at[pl.ds(my_x * m + my_y * half + c * chunk, chunk), :],
                send_sem=xs.at[c],
                recv_sem=xr.at[c],
                device_id=(other_x, my_y),
                device_id_type=pl.DeviceIdType.MESH,
            )
            r.start()
            x_rdmas.append(r)
        for t in range(T):
            c = C - T + t
            r = pltpu.make_async_remote_copy(
                src_ref=x_ref.at[pl.ds(other_y * half + c * chunk, chunk), :],
                dst_ref=out_ref.at[pl.ds(my_x * m + other_y * half + c * chunk, chunk), :],
                send_sem=xs.at[C + t],
                recv_sem=xr.at[C + t],
                device_id=(other_x, my_y),
                device_id_type=pl.DeviceIdType.MESH,
            )
            r.start()
            x_rdmas.append(r)

        y_rdmas = []
        for c in range(C - T):
            x_rdmas[c].wait_recv()
            off = other_x * m + my_y * half + c * chunk
            r = pltpu.make_async_remote_copy(
                src_ref=out_ref.at[pl.ds(off, chunk), :],
                dst_ref=out_ref.at[pl.ds(off, chunk), :],
                send_sem=ys.at[c],
                recv_sem=yr.at[c],
                device_id=(my_x, other_y),
                device_id_type=pl.DeviceIdType.MESH,
            )
            r.start()
            y_rdmas.append(r)

        own = pltpu.make_async_copy(
            x_ref, out_ref.at[pl.ds(my_x * m, m), :], own_sem
        )
        own.start()

        for c in range(C - T, C + T):
            x_rdmas[c].wait_recv()
        for c in range(C - T):
            y_rdmas[c].wait_recv()
        own.wait()
        for r in x_rdmas:
            r.wait_send()
        for r in y_rdmas:
            r.wait_send()

    return pl.pallas_call(
        body,
        out_shape=jax.ShapeDtypeStruct((2 * m, n), x.dtype),
        in_specs=[pl.BlockSpec(memory_space=pltpu.HBM)],
        out_specs=pl.BlockSpec(memory_space=pltpu.VMEM)    ,
        scratch_shapes=[
            pltpu.SemaphoreType.DMA((C + T,)),
            pltpu.SemaphoreType.DMA((C + T,)),
            pltpu.SemaphoreType.DMA((C - T,)),
            pltpu.SemaphoreType.DMA((C - T,)),
            pltpu.SemaphoreType.DMA,
        ],
        compiler_params=pltpu.CompilerParams(collective_id=0),
    )(x)
